# baseline (speedup 1.0000x reference)
import jax
import jax.numpy as jnp
from jax.experimental import pallas as pl
from jax.experimental.pallas import tpu as pltpu
from jax.experimental.pallas import tpu_sc as plsc


def _topk_body(x_ref, k_ref, idx_ref):
    keys = k_ref[...]
    kn = keys * jax.lax.rsqrt(
        jnp.maximum(jnp.sum(keys * keys, axis=1, keepdims=True), 1e-12))
    x = x_ref[...]
    xn = x * jax.lax.rsqrt(
        jnp.maximum(jnp.sum(x * x, axis=1, keepdims=True), 1e-12))
    sim = jax.lax.dot_general(
        xn.astype(jnp.bfloat16), kn.astype(jnp.bfloat16),
        (((1,), (1,)), ((), ())),
        preferred_element_type=jnp.float32)
    iota = jax.lax.broadcasted_iota(jnp.int32, sim.shape, 1)
    big = jnp.int32(2**30)
    for k in range(8):
        mx = jnp.max(sim, axis=1, keepdims=True)
        amx = jnp.min(jnp.where(sim >= mx, iota, big), axis=1)
        idx_ref[:, k] = amx
        sim = jnp.where(iota == amx[:, None], -jnp.inf, sim)


def _topk_tc(x_embed, prompt_key, block_b=256):
    B, D = x_embed.shape
    P, _ = prompt_key.shape
    idx_pad = pl.pallas_call(
        _topk_body,
        grid=(B // block_b,),
        in_specs=[
            pl.BlockSpec((block_b, D), lambda i: (i, 0)),
            pl.BlockSpec((P, D), lambda i: (0, 0)),
        ],
        out_specs=pl.BlockSpec((block_b, 128), lambda i: (i, 0)),
        out_shape=jax.ShapeDtypeStruct((B, 128), jnp.int32),
    )(x_embed, prompt_key)
    return idx_pad[:, :8]


_NW = 32
_GRP = 4
_CHUNK = _GRP * 8


def _sc_assemble(prompt, x_embed, g8, dests, B, D):
    mesh = plsc.VectorSubcoreMesh(core_axis_name="core",
                                  subcore_axis_name="subcore")
    rows_w = B // _NW
    nchunks = rows_w // _GRP
    g_per_w = rows_w * 8

    @pl.kernel(
        out_type=jax.ShapeDtypeStruct((B * 9, D), prompt.dtype),
        mesh=mesh,
        scratch_types=[
            pltpu.VMEM((g_per_w,), jnp.int32),
            pltpu.VMEM((nchunks, _CHUNK), jnp.int32),
            pltpu.VMEM((_CHUNK, D), prompt.dtype),
            pltpu.VMEM((_CHUNK, D), prompt.dtype),
            pltpu.SemaphoreType.DMA,
            pltpu.SemaphoreType.DMA,
            pltpu.SemaphoreType.DMA,
        ],
        compiler_params=pltpu.CompilerParams(use_tc_tiling_on_sc=True),
    )
    def kern(p_hbm, x_hbm, g8_hbm, d_hbm, out_hbm,
             g8_v, d_v, rows0, rows1, sem0, sem1, xsem):
        wid = (jax.lax.axis_index("subcore") * 2
               + jax.lax.axis_index("core"))
        base = wid * rows_w
        pltpu.sync_copy(g8_hbm.at[pl.ds(wid * g_per_w, g_per_w)], g8_v)
        pltpu.sync_copy(d_hbm.at[wid], d_v)
        pltpu.async_copy(
            x_hbm.at[pl.ds(base, rows_w)],
            out_hbm.at[pl.ds(8 * B + base, rows_w)], xsem)
        rows = (rows0, rows1)
        sems = (sem0, sem1)

        def start(c, b):
            pltpu.async_copy(
                p_hbm.at[g8_v.at[pl.ds(c * _CHUNK, _CHUNK)]],
                rows[b], sems[b])

        def drain(c, b):
            pltpu.make_async_copy(
                p_hbm.at[g8_v.at[pl.ds(c * _CHUNK, _CHUNK)]],
                rows[b], sems[b]).wait()

        start(0, 0)

        @pl.loop(0, nchunks, step=2)
        def _(c0):
            for b in range(2):
                c = c0 + b
                drain(c, b)

                @pl.when(c < nchunks - 1)
                def _():
                    start(c + 1, 1 - b)

                pltpu.sync_copy(rows[b], out_hbm.at[d_v.at[c]])

        pltpu.make_async_copy(
            x_hbm.at[pl.ds(base, rows_w)],
            out_hbm.at[pl.ds(8 * B + base, rows_w)], xsem).wait()

    return kern(prompt, x_embed, g8, dests)


def _dest_indices(B):
    m = jnp.arange(B * 8, dtype=jnp.int32)
    return ((m % 8) * B + m // 8).reshape(_NW, B // (_NW * _GRP), _CHUNK)


def kernel(x_embed, prompt, prompt_key):
    B, D = x_embed.shape
    idx = _topk_tc(x_embed, prompt_key)
    g8 = idx.reshape(B * 8)
    buf = _sc_assemble(prompt, x_embed, g8, _dest_indices(B), B, D)
    return buf.reshape(9, B, D).transpose(1, 0, 2)


# speedup vs baseline: 2.6574x; 2.6574x over previous
import jax
import jax.numpy as jnp
from jax.experimental import pallas as pl
from jax.experimental.pallas import tpu as pltpu
from jax.experimental.pallas import tpu_sc as plsc


def _topk_body(x_ref, k_ref, idx_ref):
    keys = k_ref[...]
    kn = keys * jax.lax.rsqrt(
        jnp.maximum(jnp.sum(keys * keys, axis=1, keepdims=True), 1e-12))
    x = x_ref[...]
    xn = x * jax.lax.rsqrt(
        jnp.maximum(jnp.sum(x * x, axis=1, keepdims=True), 1e-12))
    sim = jax.lax.dot_general(
        xn.astype(jnp.bfloat16), kn.astype(jnp.bfloat16),
        (((1,), (1,)), ((), ())),
        preferred_element_type=jnp.float32)
    iota = jax.lax.broadcasted_iota(jnp.int32, sim.shape, 1)
    big = jnp.int32(2**30)
    for k in range(8):
        mx = jnp.max(sim, axis=1, keepdims=True)
        amx = jnp.min(jnp.where(sim >= mx, iota, big), axis=1)
        idx_ref[:, k] = amx
        sim = jnp.where(iota == amx[:, None], -jnp.inf, sim)


def _topk_tc(x_embed, prompt_key, block_b=256):
    B, D = x_embed.shape
    P, _ = prompt_key.shape
    idx_pad = pl.pallas_call(
        _topk_body,
        grid=(B // block_b,),
        in_specs=[
            pl.BlockSpec((block_b, D), lambda i: (i, 0)),
            pl.BlockSpec((P, D), lambda i: (0, 0)),
        ],
        out_specs=pl.BlockSpec((block_b, 128), lambda i: (i, 0)),
        out_shape=jax.ShapeDtypeStruct((B, 128), jnp.int32),
    )(x_embed, prompt_key)
    return idx_pad[:, :8]


_NW = 32
_GRP = 4
_CHUNK = _GRP * 9


def _sc_assemble(prompt, x_embed, g8, xsrc, dests, B, D):
    mesh = plsc.VectorSubcoreMesh(core_axis_name="core",
                                  subcore_axis_name="subcore")
    rows_w = B // _NW
    nchunks = rows_w // _GRP
    g_per_w = rows_w * 8

    @pl.kernel(
        out_type=jax.ShapeDtypeStruct((B * 9, D), prompt.dtype),
        mesh=mesh,
        scratch_types=[
            pltpu.VMEM((g_per_w,), jnp.int32),
            pltpu.VMEM((nchunks, _CHUNK), jnp.int32),
            pltpu.VMEM((nchunks, _GRP), jnp.int32),
            pltpu.VMEM((_CHUNK, D), prompt.dtype),
            pltpu.VMEM((_CHUNK, D), prompt.dtype),
            pltpu.SemaphoreType.DMA,
            pltpu.SemaphoreType.DMA,
        ],
        compiler_params=pltpu.CompilerParams(use_tc_tiling_on_sc=True),
    )
    def kern(p_hbm, x_hbm, g8_hbm, ix_hbm, d_hbm, out_hbm,
             g8_v, d_v, ix_v, rows0, rows1, sem0, sem1):
        wid = (jax.lax.axis_index("subcore") * 2
               + jax.lax.axis_index("core"))
        pltpu.sync_copy(g8_hbm.at[pl.ds(wid * g_per_w, g_per_w)], g8_v)
        pltpu.sync_copy(d_hbm.at[wid], d_v)
        pltpu.sync_copy(ix_hbm.at[wid], ix_v)
        rows = (rows0, rows1)
        sems = (sem0, sem1)

        def start(c, b):
            pltpu.async_copy(
                p_hbm.at[g8_v.at[pl.ds(c * (_GRP * 8), _GRP * 8)]],
                rows[b].at[pl.ds(0, _GRP * 8)], sems[b])
            pltpu.async_copy(
                x_hbm.at[ix_v.at[c]],
                rows[b].at[pl.ds(_GRP * 8, _GRP)], sems[b])

        def drain(c, b):
            pltpu.make_async_copy(
                p_hbm.at[g8_v.at[pl.ds(c * (_GRP * 8), _GRP * 8)]],
                rows[b].at[pl.ds(0, _GRP * 8)], sems[b]).wait()
            pltpu.make_async_copy(
                x_hbm.at[ix_v.at[c]],
                rows[b].at[pl.ds(_GRP * 8, _GRP)], sems[b]).wait()

        start(0, 0)

        @pl.loop(0, nchunks, step=2)
        def _(c0):
            for b in range(2):
                c = c0 + b
                drain(c, b)

                @pl.when(c < nchunks - 1)
                def _():
                    start(c + 1, 1 - b)

                pltpu.sync_copy(rows[b], out_hbm.at[d_v.at[c]])

    return kern(prompt, x_embed, g8, xsrc, dests)


def _dest_indices(B):
    m = jnp.arange(B * 8, dtype=jnp.int32)
    d8 = ((m % 8) * B + m // 8).reshape(_NW, B // (_NW * _GRP), _GRP * 8)
    gx = jnp.arange(B, dtype=jnp.int32)
    dx = (8 * B + gx).reshape(_NW, B // (_NW * _GRP), _GRP)
    return jnp.concatenate([d8, dx], axis=2)


def _xsrc_indices(B):
    return jnp.arange(B, dtype=jnp.int32).reshape(_NW, B // (_NW * _GRP), _GRP)


def kernel(x_embed, prompt, prompt_key):
    B, D = x_embed.shape
    idx = _topk_tc(x_embed, prompt_key)
    g8 = idx.reshape(B * 8)
    buf = _sc_assemble(prompt, x_embed, g8, _xsrc_indices(B),
                       _dest_indices(B), B, D)
    return buf.reshape(9, B, D).transpose(1, 0, 2)
